# Initial kernel scaffold; baseline (speedup 1.0000x reference)
import jax
import jax.numpy as jnp
from jax.experimental import pallas as pl


def kernel(indices, emb_weight):
    raise NotImplementedError("write your pallas kernel here")


# speedup vs baseline: 1.4308x; 1.4308x over previous
import functools

import jax
import jax.numpy as jnp
from jax import lax
from jax.experimental import pallas as pl
from jax.experimental.pallas import tpu as pltpu
from jax.experimental.pallas import tpu_sc as plsc

V = 1024
D = 1024
B = 4096 * 50
NC, NS = 2, 16
NW = NC * NS
BPW = B // NW
C = 64
NCH = BPW // C


def _emb_body(idx_hbm, table_hbm, out_hbm, idx_v, rows_v, sem):
    wid = lax.axis_index("s") * NC + lax.axis_index("c")
    base = wid * BPW
    pltpu.sync_copy(idx_hbm.at[wid], idx_v)

    def chunk(j, carry):
        pltpu.async_copy(table_hbm.at[idx_v.at[j]], rows_v, sem).wait()
        pltpu.sync_copy(rows_v, out_hbm.at[pl.ds(base + j * C, C)])
        return carry

    lax.fori_loop(0, NCH, chunk, 0, unroll=False)


@jax.jit
def kernel(indices, emb_weight):
    idx = indices.reshape(NW, NCH, C).astype(jnp.int32)
    mesh = plsc.VectorSubcoreMesh(core_axis_name="c", subcore_axis_name="s")
    fn = pl.kernel(
        _emb_body,
        out_type=jax.ShapeDtypeStruct((B, D), jnp.float32),
        mesh=mesh,
        scratch_types=[
            pltpu.VMEM((NCH, C), jnp.int32),
            pltpu.VMEM((C, D), jnp.float32),
            pltpu.SemaphoreType.DMA,
        ],
    )
    out = fn(idx, emb_weight)
    return out.reshape(4096, 50, D)


# speedup vs baseline: 1.4539x; 1.0162x over previous
import functools

import jax
import jax.numpy as jnp
from jax import lax
from jax.experimental import pallas as pl
from jax.experimental.pallas import tpu as pltpu
from jax.experimental.pallas import tpu_sc as plsc

V = 1024
D = 1024
B = 4096 * 50
NC, NS = 2, 16
NW = NC * NS
BPW = B // NW
C = 40
NCH = BPW // C


def _emb_body(idx_hbm, table_hbm, out_hbm, idx_v, rows0, rows1,
              sg0, sg1, ss0, ss1):
    wid = lax.axis_index("s") * NC + lax.axis_index("c")
    base = wid * BPW
    pltpu.sync_copy(idx_hbm.at[wid], idx_v)

    def gather(j, buf, sem):
        pltpu.async_copy(table_hbm.at[idx_v.at[j]], buf, sem)

    def wait_gather(j, buf, sem):
        pltpu.make_async_copy(table_hbm.at[idx_v.at[j]], buf, sem).wait()

    def scatter(j, buf, sem):
        pltpu.async_copy(buf, out_hbm.at[pl.ds(base + j * C, C)], sem)

    def wait_scatter(buf, sem):
        pltpu.make_async_copy(buf, out_hbm.at[pl.ds(base, C)], sem).wait()

    gather(0, rows0, sg0)
    gather(1, rows1, sg1)

    def body(i, carry):
        j0 = 2 * i
        j1 = j0 + 1
        wait_gather(j0, rows0, sg0)
        scatter(j0, rows0, ss0)
        wait_gather(j1, rows1, sg1)
        scatter(j1, rows1, ss1)

        @pl.when(j0 + 2 < NCH)
        def _():
            wait_scatter(rows0, ss0)
            gather(j0 + 2, rows0, sg0)

        @pl.when(j1 + 2 < NCH)
        def _():
            wait_scatter(rows1, ss1)
            gather(j1 + 2, rows1, sg1)

        return carry

    lax.fori_loop(0, NCH // 2, body, 0, unroll=False)
    wait_scatter(rows0, ss0)
    wait_scatter(rows1, ss1)


@jax.jit
def kernel(indices, emb_weight):
    idx = indices.reshape(NW, NCH, C).astype(jnp.int32)
    mesh = plsc.VectorSubcoreMesh(core_axis_name="c", subcore_axis_name="s")
    fn = pl.kernel(
        _emb_body,
        out_type=jax.ShapeDtypeStruct((B, D), jnp.float32),
        mesh=mesh,
        scratch_types=[
            pltpu.VMEM((NCH, C), jnp.int32),
            pltpu.VMEM((C, D), jnp.float32),
            pltpu.VMEM((C, D), jnp.float32),
            pltpu.SemaphoreType.DMA,
            pltpu.SemaphoreType.DMA,
            pltpu.SemaphoreType.DMA,
            pltpu.SemaphoreType.DMA,
        ],
    )
    out = fn(idx, emb_weight)
    return out.reshape(4096, 50, D)
